# baseline (speedup 1.0000x reference)
import functools

import jax
import jax.numpy as jnp
from jax import lax
from jax.experimental import pallas as pl
from jax.experimental.pallas import tpu as pltpu
from jax.experimental.pallas import tpu_sc as plsc

N_NODES = 10000
N_EDGES = 320000
D_FEAT = 128

NC = 2
NS = 16
NW = NC * NS

N_ACC = N_NODES
CHUNK = 112
N_CHUNKS = 92
EDGES_PER_TILE = CHUNK * N_CHUNKS
E_PAD = EDGES_PER_TILE * NW
NBUF = 3

ROWS_A = 632
ROWS_B = N_ACC - 15 * ROWS_A
ROWS_B_OUT = ROWS_B


def _sc_partials(feat, src, dst):
    mesh = plsc.VectorSubcoreMesh(core_axis_name="c", subcore_axis_name="s")

    @functools.partial(
        pl.kernel,
        out_type=jax.ShapeDtypeStruct((NC, N_NODES, D_FEAT), jnp.float32),
        mesh=mesh,
        scratch_types=[
            pltpu.VMEM_SHARED((N_ACC, D_FEAT), jnp.float32),
            [pltpu.VMEM((CHUNK,), jnp.int32)] * NBUF,
            [pltpu.VMEM((CHUNK,), jnp.int32)] * NBUF,
            [pltpu.VMEM((CHUNK, D_FEAT), jnp.float32)] * NBUF,
            [pltpu.SemaphoreType.DMA] * (4 * NBUF),
        ],
    )
    def k(feat_hbm, src_hbm, dst_hbm, out_hbm,
          acc_sh, sidx, didx, rows, sems):
        c = lax.axis_index("c")
        s = lax.axis_index("s")
        wid = s * NC + c
        sem_g = sems[0:NBUF]
        sem_si = sems[NBUF:2 * NBUF]
        sem_di = sems[2 * NBUF:3 * NBUF]
        sem_sc = sems[3 * NBUF:4 * NBUF]
        ebase = wid * EDGES_PER_TILE
        row_base = s * ROWS_A

        def zbody(r, carry):
            for u in range(D_FEAT // 16):
                rows[0][r, pl.ds(u * 16, 16)] = jnp.zeros((16,), jnp.float32)
            return carry

        lax.fori_loop(0, CHUNK, zbody, 0)

        @pl.when(s < NS - 1)
        def _():
            for j in range(ROWS_A // CHUNK):
                pltpu.sync_copy(rows[0],
                                acc_sh.at[pl.ds(row_base + j * CHUNK, CHUNK)])
            rem = ROWS_A % CHUNK
            pltpu.sync_copy(
                rows[0].at[pl.ds(0, rem)],
                acc_sh.at[pl.ds(row_base + (ROWS_A // CHUNK) * CHUNK, rem)])

        @pl.when(s == NS - 1)
        def _():
            for j in range(ROWS_B // CHUNK):
                pltpu.sync_copy(rows[0],
                                acc_sh.at[pl.ds(row_base + j * CHUNK, CHUNK)])
            rem = ROWS_B % CHUNK
            pltpu.sync_copy(
                rows[0].at[pl.ds(0, rem)],
                acc_sh.at[pl.ds(row_base + (ROWS_B // CHUNK) * CHUNK, rem)])

        plsc.subcore_barrier()

        def fire_sidx(i, b):
            pltpu.async_copy(src_hbm.at[pl.ds(ebase + i * CHUNK, CHUNK)],
                             sidx[b], sem_si[b])

        def fire_didx(i, b):
            pltpu.async_copy(dst_hbm.at[pl.ds(ebase + i * CHUNK, CHUNK)],
                             didx[b], sem_di[b])

        def wait_sidx(b):
            pltpu.make_async_copy(src_hbm.at[pl.ds(0, CHUNK)],
                                  sidx[b], sem_si[b]).wait()

        def wait_didx(b):
            pltpu.make_async_copy(dst_hbm.at[pl.ds(0, CHUNK)],
                                  didx[b], sem_di[b]).wait()

        def fire_gather(b):
            pltpu.async_copy(feat_hbm.at[sidx[b]], rows[b], sem_g[b])

        def wait_gather(b):
            pltpu.make_async_copy(feat_hbm.at[sidx[b]],
                                  rows[b], sem_g[b]).wait()

        def fire_scatter(b):
            pltpu.async_copy(rows[b], acc_sh.at[didx[b]], sem_sc[b],
                             add=True)

        def wait_scatter(b):
            pltpu.make_async_copy(rows[b], acc_sh.at[didx[b]],
                                  sem_sc[b]).wait()

        def step(j, b, drain, prefetch, consume):
            b2 = (b + 2) % NBUF
            if drain:
                wait_scatter(b2)
            if prefetch:
                fire_sidx(j + 2, b2)
                fire_didx(j + 2, b2)
            if consume:
                wait_gather(b)
                wait_didx(b)
                fire_scatter(b)
            if prefetch:
                wait_sidx(b2)
                fire_gather(b2)

        for b in range(2):
            fire_sidx(b, b)
            fire_didx(b, b)
        for b in range(2):
            wait_sidx(b)
            fire_gather(b)

        step(0, 0, False, True, True)
        step(1, 1, True, True, True)
        step(2, 2, True, True, True)

        def body(g, carry):
            for u in range(NBUF):
                step(3 + g * NBUF + u, u, True, True, True)
            return carry

        lax.fori_loop(0, (N_CHUNKS - 2 - 3) // NBUF, body, 0)
        step(N_CHUNKS - 2, (N_CHUNKS - 2) % NBUF, True, False, True)
        step(N_CHUNKS - 1, (N_CHUNKS - 1) % NBUF, True, False, True)
        wait_scatter((N_CHUNKS - 1) % NBUF)

        plsc.subcore_barrier()

        @pl.when(s < NS - 1)
        def _():
            pltpu.sync_copy(acc_sh.at[pl.ds(row_base, ROWS_A)],
                            out_hbm.at[c, pl.ds(row_base, ROWS_A)])

        @pl.when(s == NS - 1)
        def _():
            pltpu.sync_copy(acc_sh.at[pl.ds(row_base, ROWS_B_OUT)],
                            out_hbm.at[c, pl.ds(row_base, ROWS_B_OUT)])

    return k(feat, src, dst)


def _combine(feat, partials):
    rows = 1000
    grid = N_NODES // rows

    def body(f_ref, a_ref, b_ref, o_ref):
        o_ref[...] = f_ref[...] + a_ref[0] + b_ref[0]

    return pl.pallas_call(
        body,
        grid=(grid,),
        in_specs=[
            pl.BlockSpec((rows, D_FEAT), lambda i: (i, 0)),
            pl.BlockSpec((1, rows, D_FEAT), lambda i: (0, i, 0)),
            pl.BlockSpec((1, rows, D_FEAT), lambda i: (1, i, 0)),
        ],
        out_specs=pl.BlockSpec((rows, D_FEAT), lambda i: (i, 0)),
        out_shape=jax.ShapeDtypeStruct((N_NODES, D_FEAT), jnp.float32),
    )(feat, partials, partials)


@jax.jit
def kernel(feat, edge_index):
    src = edge_index[0].astype(jnp.int32)
    dst = edge_index[1].astype(jnp.int32)
    n_pad = E_PAD - N_EDGES
    feat_z = jnp.concatenate([feat, jnp.zeros((8, D_FEAT), jnp.float32)])
    src = jnp.concatenate([src, jnp.full((n_pad,), N_NODES, jnp.int32)])
    dst = jnp.concatenate(
        [dst, (jnp.arange(n_pad, dtype=jnp.int32) * 13) % N_NODES])
    partials = _sc_partials(feat_z, src, dst)
    return _combine(feat, partials)
